# baseline (speedup 1.0000x reference)
import jax
import jax.numpy as jnp
from jax.experimental import pallas as pl
from jax.experimental.pallas import tpu as pltpu

T, DY, H = 8192, 256, 1024
R_W1Y, R_W2, R_B2, R_W3, R_B3 = 8192, 8448, 9472, 9480, 10504
DT = 100.0
SLOTS = (0, 0, 0, 0, 1, 1, 1, 1, 1, 1, 1, 1, 1, 2, 2, 2)
NUM_OFF = 3
TILE_B = 512
N_STEPS = 16


def _euler_kernel(idx_sref, y0_ref, slab_ref, out_ref, g0, g1, g2):
    i = pl.program_id(0)
    g = (g0, g1, g2)

    for mi in range(TILE_B):
        b = idx_sref[i * TILE_B + mi]
        for o in range(NUM_OFF):
            r = jnp.minimum(b + o, T - 1) if o else b
            c8 = pl.multiple_of((r >> 3) << 3, 8)
            chunk = slab_ref[pl.ds(c8, 8), :]
            row = pltpu.roll(chunk, -(r & 7), axis=0)[0:1, :]
            g[o][mi:mi + 1, :] = row

    w1y = slab_ref[R_W1Y:R_W1Y + DY, :]
    w2 = slab_ref[R_W2:R_W2 + H, :]
    b2 = slab_ref[R_B2:R_B2 + 1, :]
    w3 = slab_ref[R_W3:R_W3 + H, :DY]
    b3 = slab_ref[R_B3:R_B3 + 1, :DY]

    NC = 4
    HB = TILE_B // NC
    ys = [y0_ref[c * HB:(c + 1) * HB, :] for c in range(NC)]
    for k in range(N_STEPS):
        for c in range(NC):
            hxc = g[SLOTS[k]][c * HB:(c + 1) * HB, :]
            h1 = jnp.tanh(hxc + jnp.dot(ys[c], w1y,
                                        preferred_element_type=jnp.float32))
            h2 = jnp.tanh(jnp.dot(h1, w2,
                                  preferred_element_type=jnp.float32) + b2)
            ys[c] = ys[c] + DT * (jnp.dot(h2, w3,
                                          preferred_element_type=jnp.float32)
                                  + b3)

    for c in range(NC):
        out_ref[c * HB:(c + 1) * HB, :] = ys[c]


def kernel(y0, base_idx, slab):
    batch, dy = y0.shape
    assert dy == DY
    idx = base_idx.astype(jnp.int32)

    out = pl.pallas_call(
        _euler_kernel,
        out_shape=jax.ShapeDtypeStruct((batch, DY), jnp.float32),
        grid_spec=pltpu.PrefetchScalarGridSpec(
            num_scalar_prefetch=1,
            grid=(batch // TILE_B,),
            in_specs=[
                pl.BlockSpec((TILE_B, DY), lambda i, idxs: (i, 0)),
                pl.BlockSpec(slab.shape, lambda i, idxs: (0, 0)),
            ],
            out_specs=pl.BlockSpec((TILE_B, DY), lambda i, idxs: (i, 0)),
            scratch_shapes=[pltpu.VMEM((TILE_B, H), jnp.float32)
                            for _ in range(NUM_OFF)],
        ),
        compiler_params=pltpu.CompilerParams(
            dimension_semantics=("parallel",)),
    )(idx, y0, slab)
    return out


# speedup vs baseline: 1.0536x; 1.0536x over previous
import jax
import jax.numpy as jnp
from jax.experimental import pallas as pl
from jax.experimental.pallas import tpu as pltpu

T, DY, H = 8192, 256, 1024
R_W1Y, R_W2, R_B2, R_W3, R_B3 = 8192, 8448, 9472, 9480, 10504
DT = 100.0
SLOTS = (0, 0, 0, 0, 1, 1, 1, 1, 1, 1, 1, 1, 1, 2, 2, 2)
NUM_OFF = 3
TILE_B = 512
N_STEPS = 16


def _euler_kernel(idx_sref, y0_ref, slab_ref, out_ref, g0, g1, g2):
    i = pl.program_id(0)
    g = (g0, g1, g2)

    for mi in range(TILE_B):
        b = idx_sref[i * TILE_B + mi]
        for o in range(NUM_OFF):
            r = jnp.minimum(b + o, T - 1) if o else b
            c8 = pl.multiple_of((r >> 3) << 3, 8)
            chunk = slab_ref[pl.ds(c8, 8), :]
            row = pltpu.roll(chunk, -(r & 7), axis=0)[0:1, :]
            g[o][mi:mi + 1, :] = row

    w1y = slab_ref[R_W1Y:R_W1Y + DY, :]
    w2 = slab_ref[R_W2:R_W2 + H, :]
    b2 = slab_ref[R_B2:R_B2 + 1, :]
    w3 = slab_ref[R_W3:R_W3 + H, :DY]
    b3 = slab_ref[R_B3:R_B3 + 1, :DY]

    HB = TILE_B // 2
    ys = [y0_ref[0:HB, :], y0_ref[HB:TILE_B, :]]
    for k in range(N_STEPS):
        for c in range(2):
            hxc = g[SLOTS[k]][c * HB:(c + 1) * HB, :]
            h1 = jnp.tanh(hxc + jnp.dot(ys[c], w1y,
                                        preferred_element_type=jnp.float32))
            h2 = jnp.tanh(jnp.dot(h1, w2,
                                  preferred_element_type=jnp.float32) + b2)
            ys[c] = ys[c] + DT * (jnp.dot(h2, w3,
                                          preferred_element_type=jnp.float32)
                                  + b3)

    out_ref[0:HB, :] = ys[0]
    out_ref[HB:TILE_B, :] = ys[1]


def kernel(y0, base_idx, slab):
    batch, dy = y0.shape
    assert dy == DY
    idx = base_idx.astype(jnp.int32)

    out = pl.pallas_call(
        _euler_kernel,
        out_shape=jax.ShapeDtypeStruct((batch, DY), jnp.float32),
        grid_spec=pltpu.PrefetchScalarGridSpec(
            num_scalar_prefetch=1,
            grid=(batch // TILE_B,),
            in_specs=[
                pl.BlockSpec((TILE_B, DY), lambda i, idxs: (i, 0)),
                pl.BlockSpec(slab.shape, lambda i, idxs: (0, 0)),
            ],
            out_specs=pl.BlockSpec((TILE_B, DY), lambda i, idxs: (i, 0)),
            scratch_shapes=[pltpu.VMEM((TILE_B, H), jnp.float32)
                            for _ in range(NUM_OFF)],
        ),
        compiler_params=pltpu.CompilerParams(
            dimension_semantics=("parallel",)),
    )(idx, y0, slab)
    return out
